# Initial kernel scaffold; baseline (speedup 1.0000x reference)
import jax
import jax.numpy as jnp
from jax.experimental import pallas as pl


def kernel(x, edge_index, W1l, W1r, b1, W2l, W2r, b2, W3l, W3r, b3):
    raise NotImplementedError("write your pallas kernel here")


# speedup vs baseline: 3.7404x; 3.7404x over previous
import jax
import jax.numpy as jnp
from jax import lax
from jax.experimental import pallas as pl
from jax.experimental.pallas import tpu as pltpu
from jax.experimental.pallas import tpu_sc as plsc

N_NODES = 10000
N_EDGES = 320000
D_IN = 128
D_OUT = 40
D_PAD = 48

NC = 2
NS = 16
NW = NC * NS
C = 128
N_PAD = 10240
NPT = N_PAD // NS
CH_PER_TILE = 80
N_CHUNKS = NW * CH_PER_TILE
E_PAD = N_CHUNKS * C
DUMMY_DST = N_PAD - 1


def _seg_sum_sc(D, with_deg):
  mesh = plsc.VectorSubcoreMesh(
      core_axis_name="c", subcore_axis_name="s", num_cores=NC,
      num_subcores=NS)

  out_type = [jax.ShapeDtypeStruct((NC, N_PAD, D), jnp.float32)]
  scratch = [
      pltpu.VMEM((8, C), jnp.int32),
      pltpu.VMEM((8, C), jnp.int32),
      pltpu.VMEM((C, D), jnp.float32),
      pltpu.VMEM_SHARED((N_PAD, D), jnp.float32),
      pltpu.SemaphoreType.DMA,
  ]
  if with_deg:
    out_type.append(jax.ShapeDtypeStruct((NC, N_PAD, 16), jnp.float32))
    scratch += [
        pltpu.VMEM((C, 16), jnp.float32),
        pltpu.VMEM_SHARED((N_PAD, 16), jnp.float32),
    ]

  def body(*refs):
    if with_deg:
      (src2, dst2, x_hbm, z_nd, ones_h, z_n16, out, deg_out,
       sidx, didx, rows, agg_sh, sem, ones_v, deg_sh) = refs
    else:
      (src2, dst2, x_hbm, z_nd, out,
       sidx, didx, rows, agg_sh, sem) = refs
    cid = lax.axis_index("c")
    sid = lax.axis_index("s")
    wid = sid * NC + cid
    r0 = sid * NPT

    pltpu.sync_copy(z_nd.at[pl.ds(r0, NPT)], agg_sh.at[pl.ds(r0, NPT)])
    if with_deg:
      pltpu.sync_copy(z_n16.at[pl.ds(r0, NPT)], deg_sh.at[pl.ds(r0, NPT)])
      pltpu.sync_copy(ones_h, ones_v)

    plsc.subcore_barrier()

    def group(g, carry):
      base = wid * CH_PER_TILE + g * 8
      pltpu.sync_copy(src2.at[pl.ds(base, 8)], sidx)
      pltpu.sync_copy(dst2.at[pl.ds(base, 8)], didx)
      for jj in range(8):
        pltpu.async_copy(x_hbm.at[sidx.at[jj]], rows, sem).wait()
        pltpu.sync_copy(rows, agg_sh.at[didx.at[jj]], add=True)
        if with_deg:
          pltpu.sync_copy(ones_v, deg_sh.at[didx.at[jj]], add=True)
      return carry

    lax.fori_loop(0, CH_PER_TILE // 8, group, 0)
    plsc.subcore_barrier()

    pltpu.sync_copy(agg_sh.at[pl.ds(r0, NPT)], out.at[cid, pl.ds(r0, NPT)])
    if with_deg:
      pltpu.sync_copy(deg_sh.at[pl.ds(r0, NPT)],
                      deg_out.at[cid, pl.ds(r0, NPT)])

  return pl.kernel(body, out_type=out_type, mesh=mesh,
                   scratch_types=scratch,
                   compiler_params=pltpu.CompilerParams(
                       use_tc_tiling_on_sc=False))


BLK = 1000


def _tc_layer(x, a0, a1, d0, d1, Wl, Wr, b, gW=None):
  D = Wl.shape[1]
  n = x.shape[0]
  grid = (n // BLK,)
  row_spec = lambda w: pl.BlockSpec((BLK, w), lambda i: (i, 0))
  full_spec = lambda s: pl.BlockSpec(s, lambda i: (0, 0))

  def body(*refs):
    if gW is None:
      x_r, a0_r, a1_r, d0_r, d1_r, wl_r, wr_r, b_r, h_r = refs
    else:
      x_r, a0_r, a1_r, d0_r, d1_r, wl_r, wr_r, b_r, gw_r, h_r, g_r = refs
    deg = jnp.maximum(d0_r[:, 0:1] + d1_r[:, 0:1], 1.0)
    agg = (a0_r[...] + a1_r[...]) / deg
    h = (jnp.dot(agg, wl_r[...], preferred_element_type=jnp.float32)
         + jnp.dot(x_r[...], wr_r[...], preferred_element_type=jnp.float32)
         + b_r[...])
    h = jnp.maximum(h, 0.0)
    h_r[...] = h
    if gW is not None:
      g_r[...] = jnp.dot(h, gw_r[...], preferred_element_type=jnp.float32)

  in_specs = [row_spec(x.shape[1]), row_spec(D), row_spec(D),
              row_spec(16), row_spec(16),
              full_spec(Wl.shape), full_spec(Wr.shape), full_spec(b.shape)]
  out_shape = [jax.ShapeDtypeStruct((n, D), jnp.float32)]
  out_specs = [row_spec(D)]
  args = [x, a0, a1, d0, d1, Wl, Wr, b]
  if gW is not None:
    in_specs.append(full_spec(gW.shape))
    out_shape.append(jax.ShapeDtypeStruct((n, gW.shape[1]), jnp.float32))
    out_specs.append(row_spec(gW.shape[1]))
    args.append(gW)

  return pl.pallas_call(body, grid=grid, in_specs=in_specs,
                        out_specs=out_specs, out_shape=out_shape)(*args)


def _tc_final(h2, a0, a1, d0, d1, Wr, b):
  n = h2.shape[0]
  grid = (n // BLK,)
  row_spec = lambda w: pl.BlockSpec((BLK, w), lambda i: (i, 0))
  full_spec = lambda s: pl.BlockSpec(s, lambda i: (0, 0))

  def body(h_r, a0_r, a1_r, d0_r, d1_r, wr_r, b_r, o_r):
    deg = jnp.maximum(d0_r[:, 0:1] + d1_r[:, 0:1], 1.0)
    mean = (a0_r[...] + a1_r[...]) / deg
    logits = (mean
              + jnp.dot(h_r[...], wr_r[...],
                        preferred_element_type=jnp.float32)
              + b_r[...])
    m = jnp.max(logits, axis=1, keepdims=True)
    e = jnp.exp(logits - m)
    o_r[...] = (logits - m) - jnp.log(jnp.sum(e, axis=1, keepdims=True))

  return pl.pallas_call(
      body, grid=grid,
      in_specs=[row_spec(128), row_spec(D_PAD), row_spec(D_PAD),
                row_spec(16), row_spec(16),
                full_spec(Wr.shape), full_spec(b.shape)],
      out_specs=row_spec(D_PAD),
      out_shape=jax.ShapeDtypeStruct((n, D_PAD), jnp.float32))(
          h2, a0, a1, d0, d1, Wr, b)


def kernel(x, edge_index, W1l, W1r, b1, W2l, W2r, b2, W3l, W3r, b3):
  n_extra = E_PAD - N_EDGES
  src = jnp.concatenate(
      [edge_index[0].astype(jnp.int32),
       jnp.zeros((n_extra,), jnp.int32)]).reshape(N_CHUNKS, C)
  dst = jnp.concatenate(
      [edge_index[1].astype(jnp.int32),
       jnp.full((n_extra,), DUMMY_DST, jnp.int32)]).reshape(N_CHUNKS, C)

  z_nd = jnp.zeros((N_PAD, D_IN), jnp.float32)
  z_np = jnp.zeros((N_PAD, D_PAD), jnp.float32)
  z_n16 = jnp.zeros((N_PAD, 16), jnp.float32)
  ones_h = jnp.ones((C, 16), jnp.float32)

  b1r = b1.reshape(1, -1)
  b2r = b2.reshape(1, -1)
  W3lp = jnp.pad(W3l, ((0, 0), (0, D_PAD - D_OUT)))
  W3rp = jnp.pad(W3r, ((0, 0), (0, D_PAD - D_OUT)))
  b3p = jnp.concatenate(
      [b3, jnp.full((D_PAD - D_OUT,), -1e30, jnp.float32)]).reshape(1, -1)

  agg1, degp = _seg_sum_sc(D_IN, True)(src, dst, x, z_nd, ones_h, z_n16)
  d0, d1 = degp[0, :N_NODES], degp[1, :N_NODES]
  (h1,) = _tc_layer(x, agg1[0, :N_NODES], agg1[1, :N_NODES], d0, d1,
                    W1l, W1r, b1r)

  (agg2,) = _seg_sum_sc(D_IN, False)(src, dst, h1, z_nd)
  h2, g3 = _tc_layer(h1, agg2[0, :N_NODES], agg2[1, :N_NODES], d0, d1,
                     W2l, W2r, b2r, gW=W3lp)

  (agg3,) = _seg_sum_sc(D_PAD, False)(src, dst, g3, z_np)
  out = _tc_final(h2, agg3[0, :N_NODES], agg3[1, :N_NODES], d0, d1,
                  W3rp, b3p)
  return out[:, :D_OUT]
